# Initial kernel scaffold; baseline (speedup 1.0000x reference)
import jax
import jax.numpy as jnp
from jax.experimental import pallas as pl


def kernel(h, edge_index, edge_attr, fc1_w, mlp_w1, mlp_b1, mlp_w2, mlp_b2):
    raise NotImplementedError("write your pallas kernel here")


# speedup vs baseline: 2.9100x; 2.9100x over previous
import functools

import jax
import jax.numpy as jnp
from jax import lax
from jax.experimental import pallas as pl
from jax.experimental.pallas import tpu as pltpu
from jax.experimental.pallas import tpu_sc as plsc

_LOG2 = 0.6931471805599453

_NC = 2
_NS = 16
_NW = _NC * _NS
_C = 128


def _ssp(x):
    return jnp.maximum(x, 0.0) + jnp.log1p(jnp.exp(-jnp.abs(x))) - _LOG2


def _filter_body(ea_ref, w1_ref, b1_ref, w2_ref, b2_ref, out_ref):
    a = lax.dot_general(ea_ref[...], w1_ref[...], (((1,), (1,)), ((), ())),
                        preferred_element_type=jnp.float32)
    a = _ssp(a + b1_ref[...])
    b = lax.dot_general(a, w2_ref[...], (((1,), (1,)), ((), ())),
                        preferred_element_type=jnp.float32)
    out_ref[...] = _ssp(b + b2_ref[...])


def _filter_mlp(edge_attr, w1, b1, w2, b2, block_e):
    E, D = edge_attr.shape
    F = w1.shape[0]
    nb = E // block_e
    return pl.pallas_call(
        _filter_body,
        grid=(nb,),
        in_specs=[
            pl.BlockSpec((block_e, D), lambda i: (i, 0)),
            pl.BlockSpec((F, D), lambda i: (0, 0)),
            pl.BlockSpec((1, F), lambda i: (0, 0)),
            pl.BlockSpec((F, F), lambda i: (0, 0)),
            pl.BlockSpec((1, F), lambda i: (0, 0)),
        ],
        out_specs=pl.BlockSpec((block_e, F), lambda i: (i, 0)),
        out_shape=jax.ShapeDtypeStruct((E, F), jnp.float32),
    )(edge_attr, w1, b1, w2, b2)


def _hw_body(h_ref, fw_ref, out_ref):
    out_ref[...] = lax.dot_general(h_ref[...], fw_ref[...],
                                   (((1,), (1,)), ((), ())),
                                   preferred_element_type=jnp.float32)


def _node_transform(h, fc1_w, block_n):
    N, D = h.shape
    F = fc1_w.shape[0]
    nb = N // block_n
    return pl.pallas_call(
        _hw_body,
        grid=(nb,),
        in_specs=[
            pl.BlockSpec((block_n, D), lambda i: (i, 0)),
            pl.BlockSpec((F, D), lambda i: (0, 0)),
        ],
        out_specs=pl.BlockSpec((block_n, F), lambda i: (i, 0)),
        out_shape=jax.ShapeDtypeStruct((N, F), jnp.float32),
    )(h, fc1_w)


def _sc_message_scatter(hw, W, src, dst):
    N, D = hw.shape
    E = W.shape[0]
    n_chunks = E // _C
    chunks_per_tile = -(-n_chunks // _NW)
    _RPT = -(-(N // _NS) // _C) * _C
    rows_last = N - _RPT * (_NS - 1)
    assert 0 < rows_last <= _RPT and rows_last % 8 == 0

    def _pieces(total):
        out, off = [], 0
        while off < total:
            n = min(_C, total - off)
            out.append((off, n))
            off += n
        return out

    mesh = plsc.VectorSubcoreMesh(core_axis_name="c", subcore_axis_name="s",
                                  num_cores=_NC, num_subcores=_NS)

    @functools.partial(
        pl.kernel,
        out_type=jax.ShapeDtypeStruct((_NC * N, D), jnp.float32),
        mesh=mesh,
        scratch_types=[
            pltpu.VMEM((_C,), jnp.int32),
            pltpu.VMEM((_C,), jnp.int32),
            pltpu.VMEM((_C, D), jnp.float32),
            pltpu.VMEM((_C, D), jnp.float32),
            pltpu.VMEM_SHARED((N, D), jnp.float32),
            pltpu.SemaphoreType.DMA,
        ],
    )
    def sc_kernel(hw_hbm, w_hbm, src_hbm, dst_hbm, out_hbm,
                  src_v, dst_v, rows_v, w_v, acc, sem):
        c = lax.axis_index("c")
        s = lax.axis_index("s")
        wid = s * _NC + c

        def zrow(r, _):
            for cc in range(D // 16):
                w_v[r, pl.ds(cc * 16, 16)] = jnp.zeros((16,), jnp.float32)
            return 0
        lax.fori_loop(0, _C, zrow, 0)
        r0 = s * _RPT

        @pl.when(s < _NS - 1)
        def _():
            for off, n in _pieces(_RPT):
                pltpu.sync_copy(w_v.at[pl.ds(0, n)],
                                acc.at[pl.ds(r0 + off, n)])

        @pl.when(s == _NS - 1)
        def _():
            for off, n in _pieces(rows_last):
                pltpu.sync_copy(w_v.at[pl.ds(0, n)],
                                acc.at[pl.ds(r0 + off, n)])

        plsc.subcore_barrier()

        def chunk_body(j, _):
            chunk = j * _NW + wid

            @pl.when(chunk < n_chunks)
            def _():
                e0 = chunk * _C
                pltpu.sync_copy(src_hbm.at[pl.ds(e0, _C)], src_v)
                pltpu.sync_copy(dst_hbm.at[pl.ds(e0, _C)], dst_v)
                pltpu.sync_copy(w_hbm.at[pl.ds(e0, _C)], w_v)
                pltpu.async_copy(hw_hbm.at[src_v], rows_v, sem).wait()

                def mrow(r, _):
                    for cc in range(D // 16):
                        sl = pl.ds(cc * 16, 16)
                        rows_v[r, sl] = rows_v[r, sl] * w_v[r, sl]
                    return 0
                lax.fori_loop(0, _C, mrow, 0)

                pltpu.sync_copy(rows_v, acc.at[dst_v], add=True)
            return 0
        lax.fori_loop(0, chunks_per_tile, chunk_body, 0)
        plsc.subcore_barrier()

        @pl.when(s < _NS - 1)
        def _():
            pltpu.sync_copy(acc.at[pl.ds(r0, _RPT)],
                            out_hbm.at[pl.ds(c * N + r0, _RPT)])

        @pl.when(s == _NS - 1)
        def _():
            pltpu.sync_copy(acc.at[pl.ds(r0, rows_last)],
                            out_hbm.at[pl.ds(c * N + r0, rows_last)])

    return sc_kernel(hw, W, src, dst)


def _add_body(a_ref, b_ref, out_ref):
    out_ref[...] = a_ref[...] + b_ref[...]


def _add_partials(parts, N, D, block_n):
    nb = N // block_n
    off = N // block_n
    return pl.pallas_call(
        _add_body,
        grid=(nb,),
        in_specs=[
            pl.BlockSpec((block_n, D), lambda i: (i, 0)),
            pl.BlockSpec((block_n, D), lambda i: (i + off, 0)),
        ],
        out_specs=pl.BlockSpec((block_n, D), lambda i: (i, 0)),
        out_shape=jax.ShapeDtypeStruct((N, D), jnp.float32),
    )(parts, parts)


def kernel(h, edge_index, edge_attr, fc1_w, mlp_w1, mlp_b1, mlp_w2, mlp_b2):
    N, D = h.shape
    E = edge_attr.shape[0]
    ei = edge_index.astype(jnp.int32)
    src = ei[0]
    dst = ei[1]
    b1 = mlp_b1.reshape(1, -1)
    b2 = mlp_b2.reshape(1, -1)

    W = _filter_mlp(edge_attr, mlp_w1, b1, mlp_w2, b2, block_e=2000)
    hw = _node_transform(h, fc1_w, block_n=1000)
    parts = _sc_message_scatter(hw, W, src, dst)
    return _add_partials(parts, N, D, block_n=1000)
